# baseline (speedup 1.0000x reference)
import functools
import math

import jax
import jax.numpy as jnp
from jax import lax
from jax.experimental import pallas as pl
from jax.experimental.pallas import tpu as pltpu
from jax.experimental.pallas import tpu_sc as plsc

_P = 4
_M = 100000
_DLAT = 32
_DDAT = 784
_B = 1024
_HID = 256
_LV_LR = 0.01
_SIGMA = 0.01 ** 2

_NC, _NS = 2, 16
_NW = _NC * _NS
_BPW = _B // _NW
_RING = 16
_NTC = (_M + 127) // 128


def _iota16():
    return lax.iota(jnp.int32, 16)


def _vscal(ref1d, i):
    c0 = (i // 16) * 16
    v = ref1d[pl.ds(c0, 16)]
    return jnp.sum(jnp.where(_iota16() == (i - c0), v, 0))


def _blk_slice(tab, p, d):
    row = pl.multiple_of(p * _DLAT, 32)
    dal = pl.multiple_of((d // 128) * 128, 128)
    return tab.at[pl.ds(row, _DLAT), pl.ds(dal, 128)]


@functools.cache
def _sc_gather_kernel():
    mesh = plsc.VectorSubcoreMesh(core_axis_name="c", subcore_axis_name="s")

    @functools.partial(
        pl.kernel,
        out_type=(
            jax.ShapeDtypeStruct((_B * _DLAT // 128, 128), jnp.float32),
            jax.ShapeDtypeStruct((_B * _DLAT, 128), jnp.float32),
        ),
        mesh=mesh,
        scratch_types=[
            pltpu.VMEM((_BPW,), jnp.int32),
            pltpu.VMEM((_BPW,), jnp.int32),
            pltpu.VMEM((_RING, _DLAT, 128), jnp.float32),
            pltpu.VMEM((_BPW * _DLAT // 128, 128), jnp.float32),
            pltpu.SemaphoreType.DMA,
            pltpu.SemaphoreType.DMA,
        ],
        compiler_params=pltpu.CompilerParams(needs_layout_passes=False),
    )
    def _sc_gather(tab_hbm, p_hbm, d_hbm, out_hbm, blks_hbm,
                   pv, dv, blk_v, col_v, sem, wsem):
        wid = lax.axis_index("s") * _NC + lax.axis_index("c")
        base = wid * _BPW
        pltpu.sync_copy(p_hbm.at[pl.ds(base, _BPW)], pv)
        pltpu.sync_copy(d_hbm.at[pl.ds(base, _BPW)], dv)

        for r in range(_BPW // _RING):
            def _issue(j8, _):
                jj = r * _RING + j8
                p = _vscal(pv, jj)
                d = _vscal(dv, jj)
                pltpu.async_copy(_blk_slice(tab_hbm, p, d), blk_v.at[j8], sem)
                return 0

            def _wait(j8, _):
                jj = r * _RING + j8
                p = _vscal(pv, jj)
                d = _vscal(dv, jj)
                pltpu.make_async_copy(
                    _blk_slice(tab_hbm, p, d), blk_v.at[j8], sem).wait()
                return 0

            def _extract(j8, _):
                jj = r * _RING + j8
                d = _vscal(dv, jj)
                ci = jnp.broadcast_to(d % 128, (16,))
                for h in range(2):
                    rows = _iota16() + 16 * h
                    v = plsc.load_gather(blk_v.at[j8], [rows, ci])
                    fp = jj * _DLAT + 16 * h
                    plsc.store_scatter(
                        col_v,
                        [jnp.broadcast_to(fp // 128, (16,)),
                         fp % 128 + _iota16()], v)
                pltpu.async_copy(
                    blk_v.at[j8],
                    blks_hbm.at[pl.ds((base + jj) * _DLAT, _DLAT)], wsem)
                return 0

            def _stash_drain(j8, _):
                jj = r * _RING + j8
                pltpu.make_async_copy(
                    blk_v.at[j8],
                    blks_hbm.at[pl.ds((base + jj) * _DLAT, _DLAT)],
                    wsem).wait()
                return 0

            lax.fori_loop(0, _RING, _issue, 0)
            lax.fori_loop(0, _RING, _wait, 0)
            lax.fori_loop(0, _RING, _extract, 0)
            lax.fori_loop(0, _RING, _stash_drain, 0)

        pltpu.sync_copy(
            col_v, out_hbm.at[pl.ds(wid * (_BPW * _DLAT // 128),
                                    _BPW * _DLAT // 128)])

    return _sc_gather


@functools.cache
def _sc_scatter_kernel():
    mesh = plsc.VectorSubcoreMesh(core_axis_name="c", subcore_axis_name="s")

    @functools.partial(
        pl.kernel,
        out_type=(),
        mesh=mesh,
        scratch_types=[
            pltpu.VMEM((_B,), jnp.int32),
            pltpu.VMEM((_B,), jnp.int32),
            pltpu.VMEM((_B,), jnp.int32),
            pltpu.VMEM((_B * _DLAT // 128, 128), jnp.float32),
            pltpu.VMEM((_B // 128, 128), jnp.int32),
            pltpu.VMEM((_RING * _DLAT, 128), jnp.float32),
            pltpu.SemaphoreType.DMA,
        ],
        compiler_params=pltpu.CompilerParams(needs_layout_passes=False),
    )
    def _sc_scatter(tref, blks_hbm, p_hbm, d_hbm, rows_hbm, flags_hbm,
                    pv, dv, kv, rows_v, fv, blk_v, sem):
        wid = lax.axis_index("s") * _NC + lax.axis_index("c")
        base = wid * _BPW
        pltpu.sync_copy(p_hbm, pv)
        pltpu.sync_copy(d_hbm, dv)
        pltpu.sync_copy(rows_hbm, rows_v)
        pltpu.sync_copy(flags_hbm, fv)

        def _keys(c, _):
            s = pl.ds(c * 16, 16)
            kv[s] = pv[s] * _NTC + dv[s] // 128
            return 0

        lax.fori_loop(0, _B // 16, _keys, 0)

        for r in range(_BPW // _RING):
            pltpu.sync_copy(
                blks_hbm.at[pl.ds((base + r * _RING) * _DLAT,
                                  _RING * _DLAT)], blk_v)

            def _apply(j8, _):
                jj = base + r * _RING + j8
                key = _vscal(kv, jj)
                frow = jnp.broadcast_to(jj // 128, (16,))
                fcb = ((jj % 128) // 16) * 16
                f16 = plsc.load_gather(fv, [frow, fcb + _iota16()])
                flag = jnp.sum(jnp.where(_iota16() == (jj % 16), f16, 0))

                @pl.when(flag == 0)
                def _solo():
                    d = _vscal(dv, jj)
                    ci = jnp.broadcast_to(d % 128, (16,))
                    for h in range(2):
                        fp = jj * _DLAT + 16 * h
                        v = plsc.load_gather(
                            rows_v,
                            [jnp.broadcast_to(fp // 128, (16,)),
                             fp % 128 + _iota16()])
                        plsc.store_scatter(
                            blk_v.at[pl.ds(j8 * _DLAT, _DLAT)], [_iota16() + 16 * h, ci], v)

                def _chunk(c, _c):
                    k16 = kv[pl.ds(c * 16, 16)]
                    m = k16 == key
                    anym = jnp.sum(jnp.where(m, 1, 0))

                    @pl.when(anym > 0)
                    def _():
                        d16 = dv[pl.ds(c * 16, 16)]
                        for t in range(16):
                            hit = jnp.sum(
                                jnp.where(m & (_iota16() == t), 1, 0))

                            @pl.when(hit > 0)
                            def _():
                                ig = c * 16 + t
                                ci = jnp.broadcast_to(
                                    jnp.sum(jnp.where(_iota16() == t,
                                                      d16, 0)) % 128, (16,))
                                for h in range(2):
                                    fp = ig * _DLAT + 16 * h
                                    v = plsc.load_gather(
                                        rows_v,
                                        [jnp.broadcast_to(fp // 128, (16,)),
                                         fp % 128 + _iota16()])
                                    plsc.store_scatter(
                                        blk_v.at[pl.ds(j8 * _DLAT, _DLAT)],
                                        [_iota16() + 16 * h, ci], v)
                    return 0

                @pl.when(flag > 0)
                def _scan():
                    lax.fori_loop(0, _B // 16, _chunk, 0)

                return 0

            def _writeback(j8, _):
                jj = base + r * _RING + j8
                p = _vscal(pv, jj)
                d = _vscal(dv, jj)
                pltpu.async_copy(blk_v.at[pl.ds(j8 * _DLAT, _DLAT)], _blk_slice(tref, p, d), sem)
                return 0

            def _wb_drain(j8, _):
                jj = base + r * _RING + j8
                p = _vscal(pv, jj)
                d = _vscal(dv, jj)
                pltpu.make_async_copy(
                    blk_v.at[pl.ds(j8 * _DLAT, _DLAT)], _blk_slice(tref, p, d), sem).wait()
                return 0

            lax.fori_loop(0, _RING, _apply, 0)
            lax.fori_loop(0, _RING, _writeback, 0)
            lax.fori_loop(0, _RING, _wb_drain, 0)

    return _sc_scatter


def _dense_body(lv_r, data_r, noise_r, w1_r, b1_r, w2_r, b2_r, w1t_r, w2t_r,
                idxc_r, idxr_r, rows_out, loss_out, flags_out):
    lv = lv_r[...]
    h = jnp.tanh(jnp.dot(lv, w1_r[...], preferred_element_type=jnp.float32)
                 + b1_r[...])
    mu = jnp.dot(h, w2_r[...], preferred_element_type=jnp.float32) + b2_r[...]
    r = data_r[...] - mu
    inv_s2 = 1.0 / (_SIGMA * _SIGMA)

    c1 = -0.5 * _DLAT * math.log(2.0 * math.pi)
    c2 = -_DDAT * (math.log(_SIGMA) + 0.5 * math.log(2.0 * math.pi))
    total_logdens = (-0.5 * jnp.sum(lv * lv) + _B * c1
                     - 0.5 * inv_s2 * jnp.sum(r * r) + _B * c2)
    loss_out[0, 0] = -total_logdens / (_P * _B)

    dmu = r * inv_s2
    dh = (jnp.dot(dmu, w2t_r[...], preferred_element_type=jnp.float32)
          * (1.0 - h * h))
    dlv = jnp.dot(dh, w1t_r[...], preferred_element_type=jnp.float32) - lv
    upd = _LV_LR * dlv + math.sqrt(2.0 * _LV_LR) * noise_r[...]

    idxc = idxc_r[...]
    idxr = idxr_r[...]
    eq = (idxc == idxr).astype(jnp.float32)
    comb = jnp.dot(eq, upd, preferred_element_type=jnp.float32)
    rows_out[...] = lv + comb

    keyc = (idxc // _M) * _NTC + (idxc % _M) // 128
    keyr = (idxr // _M) * _NTC + (idxr % _M) // 128
    eqt = (keyc == keyr).astype(jnp.float32)
    cnt = jnp.sum(eqt, axis=1, keepdims=True)
    flags_out[...] = (cnt > 1.5).astype(jnp.int32)


def _dense(lvw, data, noise, w1, b1, w2, b2, w1t, w2t, idxc, idxr, *,
           interpret=False):
    return pl.pallas_call(
        _dense_body,
        out_shape=(
            jax.ShapeDtypeStruct((_B, _DLAT), jnp.float32),
            jax.ShapeDtypeStruct((1, 1), jnp.float32),
            jax.ShapeDtypeStruct((_B, 1), jnp.int32),
        ),
        out_specs=(
            pl.BlockSpec(memory_space=pltpu.VMEM),
            pl.BlockSpec(memory_space=pltpu.SMEM),
            pl.BlockSpec(memory_space=pltpu.VMEM),
        ),
        interpret=interpret,
    )(lvw, data, noise, w1, b1, w2, b2, w1t, w2t, idxc, idxr)


def kernel(mem, data, noise, W1, b1, W2, b2, p_idx, d_idx):
    tab = mem.transpose(0, 2, 1).reshape(_P * _DLAT, _M)
    idx = p_idx * _M + d_idx

    lvf, blks = _sc_gather_kernel()(tab, p_idx, d_idx)
    lv = lvf.reshape(_B, _DLAT)
    rows, loss, flags = _dense(lv, data, noise, W1, b1.reshape(1, _HID), W2,
                               b2.reshape(1, _DDAT), W1.T, W2.T,
                               idx.reshape(_B, 1), idx.reshape(1, _B))
    tref = jax.new_ref(tab)
    _sc_scatter_kernel()(tref, blks, p_idx, d_idx,
                         rows.reshape(_B * _DLAT // 128, 128),
                         flags.reshape(_B // 128, 128))
    out = jax.freeze(tref).reshape(_P, _DLAT, _M).transpose(0, 2, 1)
    return out, loss.reshape(())


# speedup vs baseline: 1.0215x; 1.0215x over previous
import functools
import math

import jax
import jax.numpy as jnp
from jax import lax
from jax.experimental import pallas as pl
from jax.experimental.pallas import tpu as pltpu
from jax.experimental.pallas import tpu_sc as plsc

_P = 4
_M = 100000
_DLAT = 32
_DDAT = 784
_B = 1024
_HID = 256
_LV_LR = 0.01
_SIGMA = 0.01 ** 2

_NC, _NS = 2, 16
_NW = _NC * _NS
_BPW = _B // _NW
_RING = 16
_NTC = (_M + 127) // 128


def _iota16():
    return lax.iota(jnp.int32, 16)


def _vscal(ref1d, i):
    c0 = (i // 16) * 16
    v = ref1d[pl.ds(c0, 16)]
    return jnp.sum(jnp.where(_iota16() == (i - c0), v, 0))


def _blk_slice(tab, p, d):
    row = pl.multiple_of(p * _DLAT, 32)
    dal = pl.multiple_of((d // 128) * 128, 128)
    return tab.at[pl.ds(row, _DLAT), pl.ds(dal, 128)]


@functools.cache
def _sc_gather_kernel():
    mesh = plsc.VectorSubcoreMesh(core_axis_name="c", subcore_axis_name="s")

    @functools.partial(
        pl.kernel,
        out_type=jax.ShapeDtypeStruct((_B * _DLAT // 128, 128), jnp.float32),
        mesh=mesh,
        scratch_types=[
            pltpu.VMEM((_BPW,), jnp.int32),
            pltpu.VMEM((_BPW,), jnp.int32),
            pltpu.VMEM((_RING, _DLAT, 128), jnp.float32),
            pltpu.VMEM((_BPW * _DLAT // 128, 128), jnp.float32),
            pltpu.SemaphoreType.DMA,
        ],
        compiler_params=pltpu.CompilerParams(needs_layout_passes=False),
    )
    def _sc_gather(tab_hbm, p_hbm, d_hbm, out_hbm, pv, dv, blk_v, col_v, sem):
        wid = lax.axis_index("s") * _NC + lax.axis_index("c")
        base = wid * _BPW
        pltpu.sync_copy(p_hbm.at[pl.ds(base, _BPW)], pv)
        pltpu.sync_copy(d_hbm.at[pl.ds(base, _BPW)], dv)

        for r in range(_BPW // _RING):
            def _issue(j8, _):
                jj = r * _RING + j8
                p = _vscal(pv, jj)
                d = _vscal(dv, jj)
                pltpu.async_copy(_blk_slice(tab_hbm, p, d), blk_v.at[j8], sem)
                return 0

            def _wait(j8, _):
                jj = r * _RING + j8
                p = _vscal(pv, jj)
                d = _vscal(dv, jj)
                pltpu.make_async_copy(
                    _blk_slice(tab_hbm, p, d), blk_v.at[j8], sem).wait()
                return 0

            def _extract(j8, _):
                jj = r * _RING + j8
                d = _vscal(dv, jj)
                ci = jnp.broadcast_to(d % 128, (16,))
                for h in range(2):
                    rows = _iota16() + 16 * h
                    v = plsc.load_gather(blk_v.at[j8], [rows, ci])
                    fp = jj * _DLAT + 16 * h
                    plsc.store_scatter(
                        col_v,
                        [jnp.broadcast_to(fp // 128, (16,)),
                         fp % 128 + _iota16()], v)
                return 0

            lax.fori_loop(0, _RING, _issue, 0)
            lax.fori_loop(0, _RING, _wait, 0)
            lax.fori_loop(0, _RING, _extract, 0)

        pltpu.sync_copy(
            col_v, out_hbm.at[pl.ds(wid * (_BPW * _DLAT // 128),
                                    _BPW * _DLAT // 128)])

    return _sc_gather


@functools.cache
def _sc_scatter_kernel():
    mesh = plsc.VectorSubcoreMesh(core_axis_name="c", subcore_axis_name="s")

    @functools.partial(
        pl.kernel,
        out_type=(),
        mesh=mesh,
        scratch_types=[
            pltpu.VMEM((_B,), jnp.int32),
            pltpu.VMEM((_B,), jnp.int32),
            pltpu.VMEM((_B,), jnp.int32),
            pltpu.VMEM((_B * _DLAT // 128, 128), jnp.float32),
            pltpu.VMEM((_B // 128, 128), jnp.int32),
            pltpu.VMEM((_RING, _DLAT, 128), jnp.float32),
            pltpu.SemaphoreType.DMA,
        ],
        compiler_params=pltpu.CompilerParams(needs_layout_passes=False),
    )
    def _sc_scatter(tref, tab_hbm, p_hbm, d_hbm, rows_hbm, flags_hbm,
                    pv, dv, kv, rows_v, fv, blk_v, sem):
        wid = lax.axis_index("s") * _NC + lax.axis_index("c")
        base = wid * _BPW
        pltpu.sync_copy(p_hbm, pv)
        pltpu.sync_copy(d_hbm, dv)
        pltpu.sync_copy(rows_hbm, rows_v)
        pltpu.sync_copy(flags_hbm, fv)

        def _keys(c, _):
            s = pl.ds(c * 16, 16)
            kv[s] = pv[s] * _NTC + dv[s] // 128
            return 0

        lax.fori_loop(0, _B // 16, _keys, 0)

        for r in range(_BPW // _RING):
            def _issue(j8, _):
                jj = base + r * _RING + j8
                p = _vscal(pv, jj)
                d = _vscal(dv, jj)
                pltpu.async_copy(_blk_slice(tab_hbm, p, d), blk_v.at[j8], sem)
                return 0

            def _wait(j8, _):
                jj = base + r * _RING + j8
                p = _vscal(pv, jj)
                d = _vscal(dv, jj)
                pltpu.make_async_copy(
                    _blk_slice(tab_hbm, p, d), blk_v.at[j8], sem).wait()
                return 0

            def _apply(j8, _):
                jj = base + r * _RING + j8
                key = _vscal(kv, jj)
                frow = jnp.broadcast_to(jj // 128, (16,))
                fcb = ((jj % 128) // 16) * 16
                f16 = plsc.load_gather(fv, [frow, fcb + _iota16()])
                flag = jnp.sum(jnp.where(_iota16() == (jj % 16), f16, 0))

                @pl.when(flag == 0)
                def _solo():
                    d = _vscal(dv, jj)
                    ci = jnp.broadcast_to(d % 128, (16,))
                    for h in range(2):
                        fp = jj * _DLAT + 16 * h
                        v = plsc.load_gather(
                            rows_v,
                            [jnp.broadcast_to(fp // 128, (16,)),
                             fp % 128 + _iota16()])
                        plsc.store_scatter(
                            blk_v.at[j8], [_iota16() + 16 * h, ci], v)

                def _chunk(c, _c):
                    k16 = kv[pl.ds(c * 16, 16)]
                    m = k16 == key
                    anym = jnp.sum(jnp.where(m, 1, 0))

                    @pl.when(anym > 0)
                    def _():
                        d16 = dv[pl.ds(c * 16, 16)]
                        for t in range(16):
                            hit = jnp.sum(
                                jnp.where(m & (_iota16() == t), 1, 0))

                            @pl.when(hit > 0)
                            def _():
                                ig = c * 16 + t
                                ci = jnp.broadcast_to(
                                    jnp.sum(jnp.where(_iota16() == t,
                                                      d16, 0)) % 128, (16,))
                                for h in range(2):
                                    fp = ig * _DLAT + 16 * h
                                    v = plsc.load_gather(
                                        rows_v,
                                        [jnp.broadcast_to(fp // 128, (16,)),
                                         fp % 128 + _iota16()])
                                    plsc.store_scatter(
                                        blk_v.at[j8],
                                        [_iota16() + 16 * h, ci], v)
                    return 0

                @pl.when(flag > 0)
                def _scan():
                    lax.fori_loop(0, _B // 16, _chunk, 0)

                return 0

            def _writeback(j8, _):
                jj = base + r * _RING + j8
                p = _vscal(pv, jj)
                d = _vscal(dv, jj)
                pltpu.async_copy(blk_v.at[j8], _blk_slice(tref, p, d), sem)
                return 0

            def _wb_drain(j8, _):
                jj = base + r * _RING + j8
                p = _vscal(pv, jj)
                d = _vscal(dv, jj)
                pltpu.make_async_copy(
                    blk_v.at[j8], _blk_slice(tref, p, d), sem).wait()
                return 0

            lax.fori_loop(0, _RING, _issue, 0)
            lax.fori_loop(0, _RING, _wait, 0)
            lax.fori_loop(0, _RING, _apply, 0)
            lax.fori_loop(0, _RING, _writeback, 0)
            lax.fori_loop(0, _RING, _wb_drain, 0)

    return _sc_scatter


def _dense_body(lv_r, data_r, noise_r, w1_r, b1_r, w2_r, b2_r, w1t_r, w2t_r,
                idxc_r, idxr_r, rows_out, loss_out, flags_out):
    lv = lv_r[...]
    h = jnp.tanh(jnp.dot(lv, w1_r[...], preferred_element_type=jnp.float32)
                 + b1_r[...])
    mu = jnp.dot(h, w2_r[...], preferred_element_type=jnp.float32) + b2_r[...]
    r = data_r[...] - mu
    inv_s2 = 1.0 / (_SIGMA * _SIGMA)

    c1 = -0.5 * _DLAT * math.log(2.0 * math.pi)
    c2 = -_DDAT * (math.log(_SIGMA) + 0.5 * math.log(2.0 * math.pi))
    total_logdens = (-0.5 * jnp.sum(lv * lv) + _B * c1
                     - 0.5 * inv_s2 * jnp.sum(r * r) + _B * c2)
    loss_out[0, 0] = -total_logdens / (_P * _B)

    dmu = r * inv_s2
    dh = (jnp.dot(dmu, w2t_r[...], preferred_element_type=jnp.float32)
          * (1.0 - h * h))
    dlv = jnp.dot(dh, w1t_r[...], preferred_element_type=jnp.float32) - lv
    upd = _LV_LR * dlv + math.sqrt(2.0 * _LV_LR) * noise_r[...]

    idxc = idxc_r[...]
    idxr = idxr_r[...]
    eq = (idxc == idxr).astype(jnp.float32)
    comb = jnp.dot(eq, upd, preferred_element_type=jnp.float32)
    rows_out[...] = lv + comb

    keyc = (idxc // _M) * _NTC + (idxc % _M) // 128
    keyr = (idxr // _M) * _NTC + (idxr % _M) // 128
    eqt = (keyc == keyr).astype(jnp.float32)
    cnt = jnp.sum(eqt, axis=1, keepdims=True)
    flags_out[...] = (cnt > 1.5).astype(jnp.int32)


def _dense(lvw, data, noise, w1, b1, w2, b2, w1t, w2t, idxc, idxr, *,
           interpret=False):
    return pl.pallas_call(
        _dense_body,
        out_shape=(
            jax.ShapeDtypeStruct((_B, _DLAT), jnp.float32),
            jax.ShapeDtypeStruct((1, 1), jnp.float32),
            jax.ShapeDtypeStruct((_B, 1), jnp.int32),
        ),
        out_specs=(
            pl.BlockSpec(memory_space=pltpu.VMEM),
            pl.BlockSpec(memory_space=pltpu.SMEM),
            pl.BlockSpec(memory_space=pltpu.VMEM),
        ),
        interpret=interpret,
    )(lvw, data, noise, w1, b1, w2, b2, w1t, w2t, idxc, idxr)


def kernel(mem, data, noise, W1, b1, W2, b2, p_idx, d_idx):
    tab = mem.transpose(0, 2, 1).reshape(_P * _DLAT, _M)
    idx = p_idx * _M + d_idx

    lvf = _sc_gather_kernel()(tab, p_idx, d_idx)
    lv = lvf.reshape(_B, _DLAT)
    rows, loss, flags = _dense(lv, data, noise, W1, b1.reshape(1, _HID), W2,
                               b2.reshape(1, _DDAT), W1.T, W2.T,
                               idx.reshape(_B, 1), idx.reshape(1, _B))
    tref = jax.new_ref(tab)
    _sc_scatter_kernel()(tref, tab, p_idx, d_idx,
                         rows.reshape(_B * _DLAT // 128, 128),
                         flags.reshape(_B // 128, 128))
    out = jax.freeze(tref).reshape(_P, _DLAT, _M).transpose(0, 2, 1)
    return out, loss.reshape(())


# speedup vs baseline: 1.0876x; 1.0647x over previous
import functools
import math

import jax
import jax.numpy as jnp
from jax import lax
from jax.experimental import pallas as pl
from jax.experimental.pallas import tpu as pltpu
from jax.experimental.pallas import tpu_sc as plsc

_P = 4
_M = 100000
_DLAT = 32
_DDAT = 784
_B = 1024
_HID = 256
_LV_LR = 0.01
_SIGMA = 0.01 ** 2

_NC, _NS = 2, 16
_NW = _NC * _NS
_BPW = _B // _NW
_RING = 16
_NTC = (_M + 127) // 128


def _iota16():
    return lax.iota(jnp.int32, 16)


def _vscal(ref1d, i):
    c0 = (i // 16) * 16
    v = ref1d[pl.ds(c0, 16)]
    return jnp.sum(jnp.where(_iota16() == (i - c0), v, 0))


def _blk_slice(tab, p, d):
    row = pl.multiple_of(p * _DLAT, 32)
    dal = pl.multiple_of((d // 128) * 128, 128)
    return tab.at[pl.ds(row, _DLAT), pl.ds(dal, 128)]


@functools.cache
def _sc_gather_kernel():
    mesh = plsc.VectorSubcoreMesh(core_axis_name="c", subcore_axis_name="s")

    @functools.partial(
        pl.kernel,
        out_type=jax.ShapeDtypeStruct((_B * _DLAT // 128, 128), jnp.float32),
        mesh=mesh,
        scratch_types=[
            pltpu.VMEM((_BPW,), jnp.int32),
            pltpu.VMEM((_BPW,), jnp.int32),
            pltpu.VMEM((_RING, _DLAT, 128), jnp.float32),
            pltpu.VMEM((_BPW * _DLAT // 128, 128), jnp.float32),
            pltpu.SemaphoreType.DMA,
        ],
        compiler_params=pltpu.CompilerParams(needs_layout_passes=False),
    )
    def _sc_gather(tab_hbm, p_hbm, d_hbm, out_hbm, pv, dv, blk_v, col_v, sem):
        wid = lax.axis_index("s") * _NC + lax.axis_index("c")
        base = wid * _BPW
        pltpu.sync_copy(p_hbm.at[pl.ds(base, _BPW)], pv)
        pltpu.sync_copy(d_hbm.at[pl.ds(base, _BPW)], dv)

        for r in range(_BPW // _RING):
            def _issue(j8, _):
                jj = r * _RING + j8
                p = _vscal(pv, jj)
                d = _vscal(dv, jj)
                pltpu.async_copy(_blk_slice(tab_hbm, p, d), blk_v.at[j8], sem)
                return 0

            def _wait(j8, _):
                jj = r * _RING + j8
                p = _vscal(pv, jj)
                d = _vscal(dv, jj)
                pltpu.make_async_copy(
                    _blk_slice(tab_hbm, p, d), blk_v.at[j8], sem).wait()
                return 0

            def _extract(j8, _):
                jj = r * _RING + j8
                d = _vscal(dv, jj)
                ci = jnp.broadcast_to(d % 128, (16,))
                for h in range(2):
                    rows = _iota16() + 16 * h
                    v = plsc.load_gather(blk_v.at[j8], [rows, ci])
                    fp = jj * _DLAT + 16 * h
                    plsc.store_scatter(
                        col_v,
                        [jnp.broadcast_to(fp // 128, (16,)),
                         fp % 128 + _iota16()], v)
                return 0

            lax.fori_loop(0, _RING, _issue, 0)
            lax.fori_loop(0, _RING, _wait, 0)
            lax.fori_loop(0, _RING, _extract, 0)

        pltpu.sync_copy(
            col_v, out_hbm.at[pl.ds(wid * (_BPW * _DLAT // 128),
                                    _BPW * _DLAT // 128)])

    return _sc_gather


@functools.cache
def _sc_scatter_kernel():
    mesh = plsc.VectorSubcoreMesh(core_axis_name="c", subcore_axis_name="s")

    @functools.partial(
        pl.kernel,
        out_type=(),
        mesh=mesh,
        scratch_types=[
            pltpu.VMEM((_B,), jnp.int32),
            pltpu.VMEM((_B,), jnp.int32),
            pltpu.VMEM((_B,), jnp.int32),
            pltpu.VMEM((_B * _DLAT // 128, 128), jnp.float32),
            pltpu.VMEM((_B // 128, 128), jnp.int32),
            pltpu.VMEM((_RING, _DLAT, 128), jnp.float32),
            pltpu.SemaphoreType.DMA,
            pltpu.SemaphoreType.DMA,
        ],
        compiler_params=pltpu.CompilerParams(needs_layout_passes=False),
    )
    def _sc_scatter(tref, tab_hbm, p_hbm, d_hbm, rows_hbm, flags_hbm,
                    pv, dv, kv, rows_v, fv, blk_v, sem, wsem):
        wid = lax.axis_index("s") * _NC + lax.axis_index("c")
        base = wid * _BPW
        pltpu.sync_copy(p_hbm, pv)
        pltpu.sync_copy(d_hbm, dv)
        pltpu.sync_copy(rows_hbm, rows_v)
        pltpu.sync_copy(flags_hbm, fv)

        def _keys(c, _):
            s = pl.ds(c * 16, 16)
            kv[s] = pv[s] * _NTC + dv[s] // 128
            return 0

        lax.fori_loop(0, _B // 16, _keys, 0)

        for r in range(_BPW // _RING):
            def _issue(j8, _):
                jj = base + r * _RING + j8
                p = _vscal(pv, jj)
                d = _vscal(dv, jj)
                pltpu.async_copy(_blk_slice(tab_hbm, p, d), blk_v.at[j8], sem)
                return 0

            def _apply(j8, _):
                jj = base + r * _RING + j8
                p = _vscal(pv, jj)
                d0 = _vscal(dv, jj)
                pltpu.make_async_copy(
                    _blk_slice(tab_hbm, p, d0), blk_v.at[j8], sem).wait()
                key = _vscal(kv, jj)
                frow = jnp.broadcast_to(jj // 128, (16,))
                fcb = ((jj % 128) // 16) * 16
                f16 = plsc.load_gather(fv, [frow, fcb + _iota16()])
                flag = jnp.sum(jnp.where(_iota16() == (jj % 16), f16, 0))

                @pl.when(flag == 0)
                def _solo():
                    d = _vscal(dv, jj)
                    ci = jnp.broadcast_to(d % 128, (16,))
                    for h in range(2):
                        fp = jj * _DLAT + 16 * h
                        v = plsc.load_gather(
                            rows_v,
                            [jnp.broadcast_to(fp // 128, (16,)),
                             fp % 128 + _iota16()])
                        plsc.store_scatter(
                            blk_v.at[j8], [_iota16() + 16 * h, ci], v)

                def _chunk(c, _c):
                    k16 = kv[pl.ds(c * 16, 16)]
                    m = k16 == key
                    anym = jnp.sum(jnp.where(m, 1, 0))

                    @pl.when(anym > 0)
                    def _():
                        d16 = dv[pl.ds(c * 16, 16)]
                        for t in range(16):
                            hit = jnp.sum(
                                jnp.where(m & (_iota16() == t), 1, 0))

                            @pl.when(hit > 0)
                            def _():
                                ig = c * 16 + t
                                ci = jnp.broadcast_to(
                                    jnp.sum(jnp.where(_iota16() == t,
                                                      d16, 0)) % 128, (16,))
                                for h in range(2):
                                    fp = ig * _DLAT + 16 * h
                                    v = plsc.load_gather(
                                        rows_v,
                                        [jnp.broadcast_to(fp // 128, (16,)),
                                         fp % 128 + _iota16()])
                                    plsc.store_scatter(
                                        blk_v.at[j8],
                                        [_iota16() + 16 * h, ci], v)
                    return 0

                @pl.when(flag > 0)
                def _scan():
                    lax.fori_loop(0, _B // 16, _chunk, 0)

                pltpu.async_copy(blk_v.at[j8], _blk_slice(tref, p, d0), wsem)
                return 0

            def _wb_drain(j8, _):
                jj = base + r * _RING + j8
                p = _vscal(pv, jj)
                d = _vscal(dv, jj)
                pltpu.make_async_copy(
                    blk_v.at[j8], _blk_slice(tref, p, d), wsem).wait()
                return 0

            lax.fori_loop(0, _RING, _issue, 0)
            lax.fori_loop(0, _RING, _apply, 0)
            lax.fori_loop(0, _RING, _wb_drain, 0)

    return _sc_scatter


def _dense_body(lv_r, data_r, noise_r, w1_r, b1_r, w2_r, b2_r, w1t_r, w2t_r,
                idxc_r, idxr_r, rows_out, loss_out, flags_out):
    lv = lv_r[...]
    h = jnp.tanh(jnp.dot(lv, w1_r[...], preferred_element_type=jnp.float32)
                 + b1_r[...])
    mu = jnp.dot(h, w2_r[...], preferred_element_type=jnp.float32) + b2_r[...]
    r = data_r[...] - mu
    inv_s2 = 1.0 / (_SIGMA * _SIGMA)

    c1 = -0.5 * _DLAT * math.log(2.0 * math.pi)
    c2 = -_DDAT * (math.log(_SIGMA) + 0.5 * math.log(2.0 * math.pi))
    total_logdens = (-0.5 * jnp.sum(lv * lv) + _B * c1
                     - 0.5 * inv_s2 * jnp.sum(r * r) + _B * c2)
    loss_out[0, 0] = -total_logdens / (_P * _B)

    dmu = r * inv_s2
    dh = (jnp.dot(dmu, w2t_r[...], preferred_element_type=jnp.float32)
          * (1.0 - h * h))
    dlv = jnp.dot(dh, w1t_r[...], preferred_element_type=jnp.float32) - lv
    upd = _LV_LR * dlv + math.sqrt(2.0 * _LV_LR) * noise_r[...]

    idxc = idxc_r[...]
    idxr = idxr_r[...]
    eq = (idxc == idxr).astype(jnp.float32)
    comb = jnp.dot(eq, upd, preferred_element_type=jnp.float32)
    rows_out[...] = lv + comb

    keyc = (idxc // _M) * _NTC + (idxc % _M) // 128
    keyr = (idxr // _M) * _NTC + (idxr % _M) // 128
    eqt = (keyc == keyr).astype(jnp.float32)
    cnt = jnp.sum(eqt, axis=1, keepdims=True)
    flags_out[...] = (cnt > 1.5).astype(jnp.int32)


def _dense(lvw, data, noise, w1, b1, w2, b2, w1t, w2t, idxc, idxr, *,
           interpret=False):
    return pl.pallas_call(
        _dense_body,
        out_shape=(
            jax.ShapeDtypeStruct((_B, _DLAT), jnp.float32),
            jax.ShapeDtypeStruct((1, 1), jnp.float32),
            jax.ShapeDtypeStruct((_B, 1), jnp.int32),
        ),
        out_specs=(
            pl.BlockSpec(memory_space=pltpu.VMEM),
            pl.BlockSpec(memory_space=pltpu.SMEM),
            pl.BlockSpec(memory_space=pltpu.VMEM),
        ),
        interpret=interpret,
    )(lvw, data, noise, w1, b1, w2, b2, w1t, w2t, idxc, idxr)


def kernel(mem, data, noise, W1, b1, W2, b2, p_idx, d_idx):
    tab = mem.transpose(0, 2, 1).reshape(_P * _DLAT, _M)
    idx = p_idx * _M + d_idx

    lvf = _sc_gather_kernel()(tab, p_idx, d_idx)
    lv = lvf.reshape(_B, _DLAT)
    rows, loss, flags = _dense(lv, data, noise, W1, b1.reshape(1, _HID), W2,
                               b2.reshape(1, _DDAT), W1.T, W2.T,
                               idx.reshape(_B, 1), idx.reshape(1, _B))
    tref = jax.new_ref(tab)
    _sc_scatter_kernel()(tref, tab, p_idx, d_idx,
                         rows.reshape(_B * _DLAT // 128, 128),
                         flags.reshape(_B // 128, 128))
    out = jax.freeze(tref).reshape(_P, _DLAT, _M).transpose(0, 2, 1)
    return out, loss.reshape(())
